# Initial kernel scaffold; baseline (speedup 1.0000x reference)
import jax
import jax.numpy as jnp
from jax.experimental import pallas as pl


def kernel(feature, keypoints):
    raise NotImplementedError("write your pallas kernel here")


# speedup vs baseline: 2024.8341x; 2024.8341x over previous
import functools

import jax
import jax.numpy as jnp
from jax import lax
from jax.experimental import pallas as pl
from jax.experimental.pallas import tpu as pltpu
from jax.experimental.pallas import tpu_sc as plsc

IM_FE_RATIO = 16.0

B, C, H, W = 8, 768, 32, 32
HW = H * W
N = 8192

NUM_CORES = 2
NUM_SUBCORES = 16
NW = NUM_CORES * NUM_SUBCORES
CPW = C // NW
CHUNK = 2048
NCHUNKS = N // CHUNK
GROUPS = CHUNK // 16

_f32 = jnp.float32
_i32 = jnp.int32


def _make_kernel():
    mesh = plsc.VectorSubcoreMesh(core_axis_name="c", subcore_axis_name="s")

    @functools.partial(
        pl.kernel,
        out_type=jax.ShapeDtypeStruct((B, C, N), _f32),
        mesh=mesh,
        compiler_params=pltpu.CompilerParams(needs_layout_passes=False),
        scratch_types=[
            pltpu.VMEM((CPW * HW,), _f32),
            pltpu.VMEM((CHUNK,), _f32),
            pltpu.VMEM((CHUNK,), _f32),
            pltpu.VMEM((CPW, CHUNK), _f32),
        ],
    )
    def interp(feat_hbm, kx_hbm, ky_hbm, out_hbm, tab_v, kx_v, ky_v, out_v):
        wid = lax.axis_index("s") * NUM_CORES + lax.axis_index("c")
        c0 = wid * CPW

        inv_ratio = jnp.full((16,), 1.0 / IM_FE_RATIO, _f32)
        one = jnp.full((16,), 1.0, _f32)
        half = jnp.full((16,), 0.5, _f32)
        zero = jnp.full((16,), 0.0, _f32)
        eps = jnp.full((16,), 1e-10, _f32)
        wvec = jnp.full((16,), W, _i32)
        xmax = jnp.full((16,), W - 1, _i32)
        ymax = jnp.full((16,), H - 1, _i32)
        izero = jnp.full((16,), 0, _i32)

        def batch_body(b, _):
            pltpu.sync_copy(feat_hbm.at[b, pl.ds(c0 * HW, CPW * HW)], tab_v)

            def chunk_body(j, _):
                n0 = j * CHUNK
                pltpu.sync_copy(kx_hbm.at[b, pl.ds(n0, CHUNK)], kx_v)
                pltpu.sync_copy(ky_hbm.at[b, pl.ds(n0, CHUNK)], ky_v)

                def group_body(g, _):
                    xs = kx_v[pl.ds(g * 16, 16)] * inv_ratio
                    ys = ky_v[pl.ds(g * 16, 16)] * inv_ratio
                    fx = jnp.maximum(xs.astype(_i32), izero)
                    fy = jnp.maximum(ys.astype(_i32), izero)
                    fxf = fx.astype(_f32)
                    fyf = fy.astype(_f32)
                    cx = jnp.minimum(
                        jnp.where(xs > fxf, fx + 1, fx), xmax)
                    cy = jnp.minimum(
                        jnp.where(ys > fyf, fy + 1, fy), ymax)
                    ux = xs - fxf
                    uy = ys - fyf
                    mask = (jnp.where(xs > eps, half, zero)
                            + jnp.where(ys > eps, half, zero))
                    lxm = (one - ux) * mask
                    uxm = ux * mask
                    ly = one - uy
                    w00 = lxm * ly
                    w01 = uxm * ly
                    w10 = lxm * uy
                    w11 = uxm * uy
                    rf = fy * wvec
                    rc = cy * wvec
                    i00 = rf + fx
                    i01 = rf + cx
                    i10 = rc + fx
                    i11 = rc + cx
                    for c in range(CPW):
                        cb = jnp.full((16,), c * HW, _i32)
                        acc = plsc.load_gather(tab_v, [i00 + cb]) * w00
                        acc = acc + plsc.load_gather(tab_v, [i01 + cb]) * w01
                        acc = acc + plsc.load_gather(tab_v, [i10 + cb]) * w10
                        acc = acc + plsc.load_gather(tab_v, [i11 + cb]) * w11
                        out_v[c, pl.ds(g * 16, 16)] = acc
                    return 0

                lax.fori_loop(0, GROUPS, group_body, 0)
                pltpu.sync_copy(
                    out_v, out_hbm.at[b, pl.ds(c0, CPW), pl.ds(n0, CHUNK)])
                return 0

            lax.fori_loop(0, NCHUNKS, chunk_body, 0)
            return 0

        lax.fori_loop(0, B, batch_body, 0)

    return interp


_interp = _make_kernel()


def kernel(feature, keypoints):
    feat = feature.reshape(B, C * HW)
    kx = keypoints[:, :, 0]
    ky = keypoints[:, :, 1]
    return _interp(feat, kx, ky)
